# baseline (speedup 1.0000x reference)
import functools

import jax
import jax.numpy as jnp
from jax import lax
from jax.experimental import pallas as pl
from jax.experimental.pallas import tpu as pltpu
from jax.experimental.pallas import tpu_sc as plsc

D = 128
MINUTE_SIZE = 4
HOUR_SIZE = 24
WEEKDAY = 7
WD_PAD = 8
T_ROWS = MINUTE_SIZE * HOUR_SIZE
F_ROWS = T_ROWS * WD_PAD

NC, NS, L = 2, 16, 16
NW = NC * NS
CHUNK = 64
NBUF = 8
LA = 4
HBM_BS = (3, 7)


def _tc_table(minute_embed, hour_embed, weekday_embed):
    def body(m_ref, h_ref, w_ref, out_ref):
        r = lax.broadcasted_iota(jnp.int32, (F_ROWS, 1), 0)
        hour_id = r // (MINUTE_SIZE * WD_PAD)
        min_id = (r // WD_PAD) % MINUTE_SIZE
        wd_id = r % WD_PAD
        h_sel = jnp.zeros((F_ROWS, D), jnp.float32)
        for k in range(HOUR_SIZE):
            h_sel = jnp.where(hour_id == k, h_ref[k, :][None, :], h_sel)
        m_sel = jnp.zeros((F_ROWS, D), jnp.float32)
        for k in range(MINUTE_SIZE):
            m_sel = jnp.where(min_id == k, m_ref[k, :][None, :], m_sel)
        w_sel = jnp.zeros((F_ROWS, D), jnp.float32)
        for k in range(WEEKDAY):
            w_sel = jnp.where(wd_id == k, w_ref[k, :][None, :], w_sel)
        out_ref[...] = h_sel + m_sel + w_sel

    return pl.pallas_call(
        body,
        out_shape=jax.ShapeDtypeStruct((F_ROWS, D), jnp.float32),
    )(minute_embed, hour_embed, weekday_embed)


def _make_sc_gather(b_total):
    rows_pw = b_total // NW
    n_chunks = rows_pw // CHUNK
    assert n_chunks % NBUF == 0
    assert CHUNK % L == 0

    mesh = plsc.VectorSubcoreMesh(
        core_axis_name="c", subcore_axis_name="s", num_cores=NC, num_subcores=NS
    )

    @functools.partial(
        pl.kernel,
        out_type=jax.ShapeDtypeStruct((b_total, D), jnp.float32),
        mesh=mesh,
        scratch_types=[
            pltpu.VMEM_SHARED((F_ROWS, D), jnp.float32),
            pltpu.VMEM((rows_pw,), jnp.int32),
            pltpu.VMEM((rows_pw,), jnp.int32),
        ]
        + [pltpu.VMEM((CHUNK, D), jnp.float32)] * NBUF
        + [pltpu.SemaphoreType.DMA] * (2 * NBUF + 2),
    )
    def sc_gather(table_hbm, time_hbm, wd_hbm, out_hbm,
                  table_sh, idx_all, wd_all, *scratch):
        rbufs = scratch[:NBUF]
        sg = scratch[NBUF:2 * NBUF]
        so = scratch[2 * NBUF:2 * NBUF + NBUF]
        st, sw = scratch[3 * NBUF], scratch[3 * NBUF + 1]
        cid = lax.axis_index("c")
        sid = lax.axis_index("s")
        wid = sid * NC + cid
        base = wid * rows_pw

        pltpu.async_copy(time_hbm.at[pl.ds(base, rows_pw)], idx_all, st)
        pltpu.async_copy(wd_hbm.at[pl.ds(base, rows_pw)], wd_all, sw)

        @pl.when(sid == 0)
        def _():
            pltpu.sync_copy(table_hbm, table_sh)

        pltpu.make_async_copy(time_hbm.at[pl.ds(base, rows_pw)], idx_all, st).wait()
        pltpu.make_async_copy(wd_hbm.at[pl.ds(base, rows_pw)], wd_all, sw).wait()

        def fuse_idx(i):
            for j in range(CHUNK // L):
                s = pl.ds(i * CHUNK + j * L, L)
                idx_all[s] = idx_all[s] * WD_PAD + wd_all[s]

        def idx_ref(i):
            return idx_all.at[pl.ds(i * CHUNK, CHUNK)]

        def out_slice(i):
            return out_hbm.at[pl.ds(base + i * CHUNK, CHUNK)]

        plsc.subcore_barrier()

        def src(b):
            return table_hbm if b in HBM_BS else table_sh

        for i in range(LA):
            fuse_idx(i)
            pltpu.async_copy(src(i).at[idx_ref(i)], rbufs[i], sg[i])

        def group(g, _):
            for b in range(NBUF):
                i = g * NBUF + b
                b2 = (b + LA) % NBUF

                @pl.when(i >= NBUF - LA)
                def _():
                    pltpu.make_async_copy(
                        rbufs[b2], out_slice(i - (NBUF - LA)), so[b2]
                    ).wait()

                @pl.when(i + LA < n_chunks)
                def _():
                    fuse_idx(i + LA)
                    pltpu.async_copy(src(b2).at[idx_ref(i + LA)], rbufs[b2], sg[b2])

                pltpu.make_async_copy(src(b).at[idx_ref(i)], rbufs[b], sg[b]).wait()
                pltpu.async_copy(rbufs[b], out_slice(i), so[b])
            return 0

        lax.fori_loop(0, n_chunks // NBUF, group, 0)

        for k in range(NBUF - LA, 0, -1):
            i = n_chunks - k
            pltpu.make_async_copy(rbufs[i % NBUF], out_slice(i), so[i % NBUF]).wait()

    return sc_gather


def kernel(time, weekday, minute_embed, hour_embed, weekday_embed):
    orig_shape = time.shape
    b_total = time.size
    table = _tc_table(minute_embed, hour_embed, weekday_embed)
    out = _make_sc_gather(b_total)(
        table, time.reshape(-1), weekday.reshape(-1)
    )
    return out.reshape(*orig_shape, D)


# speedup vs baseline: 1.5587x; 1.5587x over previous
import functools

import jax
import jax.numpy as jnp
from jax import lax
from jax.experimental import pallas as pl
from jax.experimental.pallas import tpu as pltpu
from jax.experimental.pallas import tpu_sc as plsc

D = 128
MINUTE_SIZE = 4
HOUR_SIZE = 24
WEEKDAY = 7
WD_PAD = 8
T_ROWS = MINUTE_SIZE * HOUR_SIZE
F_ROWS = T_ROWS * WD_PAD

NC, NS, L = 2, 16, 16
NW = NC * NS
HC = 2
NBUF = 4
LA = 2
HBM_BS = ()


def _tc_prep(time, weekday, minute_embed, hour_embed, weekday_embed):
    def body(t_ref, wd_ref, m_ref, h_ref, w_ref, out_ref, idx_ref):
        r = lax.broadcasted_iota(jnp.int32, (F_ROWS, 1), 0)
        hour_id = r // (MINUTE_SIZE * WD_PAD)
        min_id = (r // WD_PAD) % MINUTE_SIZE
        wd_id = r % WD_PAD
        h_sel = jnp.zeros((F_ROWS, D), jnp.float32)
        for k in range(HOUR_SIZE):
            h_sel = jnp.where(hour_id == k, h_ref[k, :][None, :], h_sel)
        m_sel = jnp.zeros((F_ROWS, D), jnp.float32)
        for k in range(MINUTE_SIZE):
            m_sel = jnp.where(min_id == k, m_ref[k, :][None, :], m_sel)
        w_sel = jnp.zeros((F_ROWS, D), jnp.float32)
        for k in range(WEEKDAY):
            w_sel = jnp.where(wd_id == k, w_ref[k, :][None, :], w_sel)
        out_ref[...] = h_sel + m_sel + w_sel
        idx_ref[...] = (t_ref[...] * WD_PAD + wd_ref[...]).reshape(-1)

    return pl.pallas_call(
        body,
        out_shape=(
            jax.ShapeDtypeStruct((F_ROWS, D), jnp.float32),
            jax.ShapeDtypeStruct((time.size,), jnp.int32),
        ),
    )(time, weekday, minute_embed, hour_embed, weekday_embed)


def _make_sc_gather(batch, t_len, tp_len):
    rows_b = batch // NW
    rows_pw = rows_b * t_len
    idx_pw = rows_b * tp_len
    ch0 = -(-(t_len // 2) // 8) * 8
    ch1 = t_len - ch0
    chs = (ch0, ch1)
    n_chunks = rows_b * HC
    assert n_chunks % NBUF == 0 and NBUF % HC == 0 and 0 < ch1

    mesh = plsc.VectorSubcoreMesh(
        core_axis_name="c", subcore_axis_name="s", num_cores=NC, num_subcores=NS
    )

    @functools.partial(
        pl.kernel,
        out_type=jax.ShapeDtypeStruct((batch * t_len, D), jnp.float32),
        mesh=mesh,
        scratch_types=[
            pltpu.VMEM_SHARED((F_ROWS, D), jnp.float32),
            pltpu.VMEM((idx_pw,), jnp.int32),
        ]
        + [pltpu.VMEM((chs[b % HC], D), jnp.float32) for b in range(NBUF)]
        + [pltpu.SemaphoreType.DMA] * (2 * NBUF + 1),
    )
    def sc_gather(table_hbm, idx_hbm, out_hbm, table_sh, idx_all, *scratch):
        rbufs = scratch[:NBUF]
        sg = scratch[NBUF:2 * NBUF]
        so = scratch[2 * NBUF:2 * NBUF + NBUF]
        st = scratch[3 * NBUF]
        cid = lax.axis_index("c")
        sid = lax.axis_index("s")
        wid = sid * NC + cid
        base = wid * rows_pw

        pltpu.async_copy(idx_hbm.at[pl.ds(wid * idx_pw, idx_pw)], idx_all, st)

        @pl.when(sid == 0)
        def _():
            pltpu.sync_copy(table_hbm, table_sh)

        pltpu.make_async_copy(idx_hbm.at[pl.ds(wid * idx_pw, idx_pw)], idx_all, st).wait()

        def idx_ref(i, b):
            off = 0 if b % HC == 0 else ch0
            return idx_all.at[pl.ds((i // HC) * tp_len + off, chs[b % HC])]

        def out_slice(i, b):
            off = 0 if b % HC == 0 else ch0
            return out_hbm.at[pl.ds(base + (i // HC) * t_len + off, chs[b % HC])]

        plsc.subcore_barrier()

        def src(b):
            return table_hbm if b in HBM_BS else table_sh

        for i in range(LA):
            pltpu.async_copy(src(i).at[idx_ref(i, i)], rbufs[i], sg[i])

        def group(g, _):
            for b in range(NBUF):
                i = g * NBUF + b
                b2 = (b + LA) % NBUF

                @pl.when(i >= NBUF - LA)
                def _():
                    pltpu.make_async_copy(
                        rbufs[b2], out_slice(i - (NBUF - LA), b2), so[b2]
                    ).wait()

                @pl.when(i + LA < n_chunks)
                def _():
                    pltpu.async_copy(src(b2).at[idx_ref(i + LA, b2)], rbufs[b2], sg[b2])

                pltpu.make_async_copy(src(b).at[idx_ref(i, b)], rbufs[b], sg[b]).wait()
                pltpu.async_copy(rbufs[b], out_slice(i, b), so[b])
            return 0

        lax.fori_loop(0, n_chunks // NBUF, group, 0)

        for k in range(NBUF - LA, 0, -1):
            i = n_chunks - k
            pltpu.make_async_copy(
                rbufs[i % NBUF], out_slice(i, i % NBUF), so[i % NBUF]
            ).wait()

    return sc_gather


def kernel(time, weekday, minute_embed, hour_embed, weekday_embed):
    batch, t_len = time.shape
    tp_len = -(-t_len // D) * D
    pad = ((0, 0), (0, tp_len - t_len))
    time_p = jnp.pad(time, pad)
    weekday_p = jnp.pad(weekday, pad)
    table, fused_idx = _tc_prep(time_p, weekday_p, minute_embed, hour_embed,
                                weekday_embed)
    out = _make_sc_gather(batch, t_len, tp_len)(table, fused_idx)
    return out.reshape(batch, t_len, D)


# speedup vs baseline: 1.6148x; 1.0360x over previous
import functools

import jax
import jax.numpy as jnp
from jax import lax
from jax.experimental import pallas as pl
from jax.experimental.pallas import tpu as pltpu
from jax.experimental.pallas import tpu_sc as plsc

D = 128
MINUTE_SIZE = 4
HOUR_SIZE = 24
WEEKDAY = 7
WD_PAD = 8
T_ROWS = MINUTE_SIZE * HOUR_SIZE
F_ROWS = T_ROWS * WD_PAD

NC, NS, L = 2, 16, 16
NW = NC * NS
HC = 2
NBUF = 4
LA = 2
HBM_BS = ()


def _tc_prep(time, weekday, minute_embed, hour_embed, weekday_embed):
    batch, t_len = time.shape
    tp_len = -(-t_len // D) * D

    def body(t_ref, wd_ref, m_ref, h_ref, w_ref, out_ref, idx_ref):
        r = lax.broadcasted_iota(jnp.int32, (F_ROWS, 1), 0)
        hour_id = r // (MINUTE_SIZE * WD_PAD)
        min_id = (r // WD_PAD) % MINUTE_SIZE
        wd_id = r % WD_PAD
        h_sel = jnp.zeros((F_ROWS, D), jnp.float32)
        for k in range(HOUR_SIZE):
            h_sel = jnp.where(hour_id == k, h_ref[k, :][None, :], h_sel)
        m_sel = jnp.zeros((F_ROWS, D), jnp.float32)
        for k in range(MINUTE_SIZE):
            m_sel = jnp.where(min_id == k, m_ref[k, :][None, :], m_sel)
        w_sel = jnp.zeros((F_ROWS, D), jnp.float32)
        for k in range(WEEKDAY):
            w_sel = jnp.where(wd_id == k, w_ref[k, :][None, :], w_sel)
        out_ref[...] = h_sel + m_sel + w_sel
        idx_ref[...] = (t_ref[...] * WD_PAD + wd_ref[...]).reshape(-1)

    full = lambda shape: pl.BlockSpec(shape, lambda i: (0,) * len(shape))
    return pl.pallas_call(
        body,
        grid=(1,),
        in_specs=[
            full((batch, tp_len)),
            full((batch, tp_len)),
            full((MINUTE_SIZE, D)),
            full((HOUR_SIZE, D)),
            full((WEEKDAY, D)),
        ],
        out_specs=(full((F_ROWS, D)), full((batch * tp_len,))),
        out_shape=(
            jax.ShapeDtypeStruct((F_ROWS, D), jnp.float32),
            jax.ShapeDtypeStruct((batch * tp_len,), jnp.int32),
        ),
    )(time, weekday, minute_embed, hour_embed, weekday_embed)


def _make_sc_gather(batch, t_len, tp_len):
    rows_b = batch // NW
    rows_pw = rows_b * t_len
    idx_pw = rows_b * tp_len
    ch0 = -(-(t_len // 2) // 8) * 8
    ch1 = t_len - ch0
    chs = (ch0, ch1)
    n_chunks = rows_b * HC
    assert n_chunks % NBUF == 0 and NBUF % HC == 0 and 0 < ch1

    mesh = plsc.VectorSubcoreMesh(
        core_axis_name="c", subcore_axis_name="s", num_cores=NC, num_subcores=NS
    )

    @functools.partial(
        pl.kernel,
        out_type=jax.ShapeDtypeStruct((batch * t_len, D), jnp.float32),
        mesh=mesh,
        scratch_types=[
            pltpu.VMEM_SHARED((F_ROWS, D), jnp.float32),
            pltpu.VMEM((idx_pw,), jnp.int32),
        ]
        + [pltpu.VMEM((chs[b % HC], D), jnp.float32) for b in range(NBUF)]
        + [pltpu.SemaphoreType.DMA] * (2 * NBUF + 1),
    )
    def sc_gather(table_hbm, idx_hbm, out_hbm, table_sh, idx_all, *scratch):
        rbufs = scratch[:NBUF]
        sg = scratch[NBUF:2 * NBUF]
        so = scratch[2 * NBUF:2 * NBUF + NBUF]
        st = scratch[3 * NBUF]
        cid = lax.axis_index("c")
        sid = lax.axis_index("s")
        wid = sid * NC + cid
        base = wid * rows_pw

        pltpu.async_copy(idx_hbm.at[pl.ds(wid * idx_pw, idx_pw)], idx_all, st)

        @pl.when(sid == 0)
        def _():
            pltpu.sync_copy(table_hbm, table_sh)

        pltpu.make_async_copy(idx_hbm.at[pl.ds(wid * idx_pw, idx_pw)], idx_all, st).wait()

        def idx_ref(i, b):
            off = 0 if b % HC == 0 else ch0
            return idx_all.at[pl.ds((i // HC) * tp_len + off, chs[b % HC])]

        def out_slice(i, b):
            off = 0 if b % HC == 0 else ch0
            return out_hbm.at[pl.ds(base + (i // HC) * t_len + off, chs[b % HC])]

        plsc.subcore_barrier()

        def src(b):
            return table_hbm if b in HBM_BS else table_sh

        for i in range(LA):
            pltpu.async_copy(src(i).at[idx_ref(i, i)], rbufs[i], sg[i])

        def group(g, _):
            for b in range(NBUF):
                i = g * NBUF + b
                b2 = (b + LA) % NBUF

                @pl.when(i >= NBUF - LA)
                def _():
                    pltpu.make_async_copy(
                        rbufs[b2], out_slice(i - (NBUF - LA), b2), so[b2]
                    ).wait()

                @pl.when(i + LA < n_chunks)
                def _():
                    pltpu.async_copy(src(b2).at[idx_ref(i + LA, b2)], rbufs[b2], sg[b2])

                pltpu.make_async_copy(src(b).at[idx_ref(i, b)], rbufs[b], sg[b]).wait()
                pltpu.async_copy(rbufs[b], out_slice(i, b), so[b])
            return 0

        lax.fori_loop(0, n_chunks // NBUF, group, 0)

        for k in range(NBUF - LA, 0, -1):
            i = n_chunks - k
            pltpu.make_async_copy(
                rbufs[i % NBUF], out_slice(i, i % NBUF), so[i % NBUF]
            ).wait()

    return sc_gather


def kernel(time, weekday, minute_embed, hour_embed, weekday_embed):
    batch, t_len = time.shape
    tp_len = -(-t_len // D) * D
    table, fused_idx = _tc_prep(time, weekday, minute_embed, hour_embed,
                                weekday_embed)
    out = _make_sc_gather(batch, t_len, tp_len)(table, fused_idx)
    return out.reshape(batch, t_len, D)


# speedup vs baseline: 1.6260x; 1.0069x over previous
import functools

import jax
import jax.numpy as jnp
from jax import lax
from jax.experimental import pallas as pl
from jax.experimental.pallas import tpu as pltpu
from jax.experimental.pallas import tpu_sc as plsc

D = 128
MINUTE_SIZE = 4
HOUR_SIZE = 24
WEEKDAY = 7
WD_PAD = 8
T_ROWS = MINUTE_SIZE * HOUR_SIZE
F_ROWS = T_ROWS * WD_PAD

NC, NS, L = 2, 16, 16
NW = NC * NS
HC = 2
NBUF = 4
LA = 3
HBM_BS = ()


def _tc_prep(time, weekday, minute_embed, hour_embed, weekday_embed):
    batch, t_len = time.shape
    tp_len = -(-t_len // D) * D

    def body(t_ref, wd_ref, m_ref, h_ref, w_ref, out_ref, idx_ref):
        r = lax.broadcasted_iota(jnp.int32, (F_ROWS, 1), 0)
        hour_id = r // (MINUTE_SIZE * WD_PAD)
        min_id = (r // WD_PAD) % MINUTE_SIZE
        wd_id = r % WD_PAD
        h_sel = jnp.zeros((F_ROWS, D), jnp.float32)
        for k in range(HOUR_SIZE):
            h_sel = jnp.where(hour_id == k, h_ref[k, :][None, :], h_sel)
        m_sel = jnp.zeros((F_ROWS, D), jnp.float32)
        for k in range(MINUTE_SIZE):
            m_sel = jnp.where(min_id == k, m_ref[k, :][None, :], m_sel)
        w_sel = jnp.zeros((F_ROWS, D), jnp.float32)
        for k in range(WEEKDAY):
            w_sel = jnp.where(wd_id == k, w_ref[k, :][None, :], w_sel)
        out_ref[...] = h_sel + m_sel + w_sel
        idx_ref[...] = (t_ref[...] * WD_PAD + wd_ref[...]).reshape(-1)

    full = lambda shape: pl.BlockSpec(shape, lambda i: (0,) * len(shape))
    return pl.pallas_call(
        body,
        grid=(1,),
        in_specs=[
            full((batch, tp_len)),
            full((batch, tp_len)),
            full((MINUTE_SIZE, D)),
            full((HOUR_SIZE, D)),
            full((WEEKDAY, D)),
        ],
        out_specs=(full((F_ROWS, D)), full((batch * tp_len,))),
        out_shape=(
            jax.ShapeDtypeStruct((F_ROWS, D), jnp.float32),
            jax.ShapeDtypeStruct((batch * tp_len,), jnp.int32),
        ),
    )(time, weekday, minute_embed, hour_embed, weekday_embed)


def _make_sc_gather(batch, t_len, tp_len):
    rows_b = batch // NW
    rows_pw = rows_b * t_len
    idx_pw = rows_b * tp_len
    ch0 = -(-(t_len // 2) // 8) * 8
    ch1 = t_len - ch0
    chs = (ch0, ch1)
    n_chunks = rows_b * HC
    assert n_chunks % NBUF == 0 and NBUF % HC == 0 and 0 < ch1

    mesh = plsc.VectorSubcoreMesh(
        core_axis_name="c", subcore_axis_name="s", num_cores=NC, num_subcores=NS
    )

    @functools.partial(
        pl.kernel,
        out_type=jax.ShapeDtypeStruct((batch * t_len, D), jnp.float32),
        mesh=mesh,
        scratch_types=[
            pltpu.VMEM_SHARED((F_ROWS, D), jnp.float32),
            pltpu.VMEM((idx_pw,), jnp.int32),
        ]
        + [pltpu.VMEM((chs[b % HC], D), jnp.float32) for b in range(NBUF)]
        + [pltpu.SemaphoreType.DMA] * (2 * NBUF + 1),
    )
    def sc_gather(table_hbm, idx_hbm, out_hbm, table_sh, idx_all, *scratch):
        rbufs = scratch[:NBUF]
        sg = scratch[NBUF:2 * NBUF]
        so = scratch[2 * NBUF:2 * NBUF + NBUF]
        st = scratch[3 * NBUF]
        cid = lax.axis_index("c")
        sid = lax.axis_index("s")
        wid = sid * NC + cid
        base = wid * rows_pw

        pltpu.async_copy(idx_hbm.at[pl.ds(wid * idx_pw, idx_pw)], idx_all, st)

        @pl.when(sid == 0)
        def _():
            pltpu.sync_copy(table_hbm, table_sh)

        pltpu.make_async_copy(idx_hbm.at[pl.ds(wid * idx_pw, idx_pw)], idx_all, st).wait()

        def idx_ref(i, b):
            off = 0 if b % HC == 0 else ch0
            return idx_all.at[pl.ds((i // HC) * tp_len + off, chs[b % HC])]

        def out_slice(i, b):
            off = 0 if b % HC == 0 else ch0
            return out_hbm.at[pl.ds(base + (i // HC) * t_len + off, chs[b % HC])]

        plsc.subcore_barrier()

        def src(b):
            return table_hbm if b in HBM_BS else table_sh

        for i in range(LA):
            pltpu.async_copy(src(i).at[idx_ref(i, i)], rbufs[i], sg[i])

        def group(g, _):
            for b in range(NBUF):
                i = g * NBUF + b
                b2 = (b + LA) % NBUF

                @pl.when(i >= NBUF - LA)
                def _():
                    pltpu.make_async_copy(
                        rbufs[b2], out_slice(i - (NBUF - LA), b2), so[b2]
                    ).wait()

                @pl.when(i + LA < n_chunks)
                def _():
                    pltpu.async_copy(src(b2).at[idx_ref(i + LA, b2)], rbufs[b2], sg[b2])

                pltpu.make_async_copy(src(b).at[idx_ref(i, b)], rbufs[b], sg[b]).wait()
                pltpu.async_copy(rbufs[b], out_slice(i, b), so[b])
            return 0

        lax.fori_loop(0, n_chunks // NBUF, group, 0)

        for k in range(NBUF - LA, 0, -1):
            i = n_chunks - k
            pltpu.make_async_copy(
                rbufs[i % NBUF], out_slice(i, i % NBUF), so[i % NBUF]
            ).wait()

    return sc_gather


def kernel(time, weekday, minute_embed, hour_embed, weekday_embed):
    batch, t_len = time.shape
    tp_len = -(-t_len // D) * D
    table, fused_idx = _tc_prep(time, weekday, minute_embed, hour_embed,
                                weekday_embed)
    out = _make_sc_gather(batch, t_len, tp_len)(table, fused_idx)
    return out.reshape(batch, t_len, D)
